# baseline (speedup 1.0000x reference)
import functools

import jax
import jax.numpy as jnp
from jax import lax
from jax.experimental import pallas as pl
from jax.experimental.pallas import tpu as pltpu
from jax.experimental.pallas import tpu_sc as plsc

N = 10000
D = 128
NP = 10112
E = 320000
NC, NS, L = 2, 16, 16
NW = NC * NS
KE = 64
CH = 160
EPT = CH * KE
EPAD = EPT * NW
STRIPE = NP // NS
NBUF = 4
NIDX = 8
CH0 = 304
CH1 = 16
EPT0 = CH0 * KE
EPT1 = CH1 * KE
F0 = EPT0 * NS


@functools.cache
def _mesh():
    return plsc.VectorSubcoreMesh(
        core_axis_name="c", subcore_axis_name="s", num_cores=NC, num_subcores=NS)


def _deg_body(colp_hbm, out_hbm, col4, ones_v, zbuf, si0, si1, si2, si3,
              ss0, ss1, ss2, ss3, acc):
    si = (si0, si1, si2, si3)
    ss = (ss0, ss1, ss2, ss3)
    cid = lax.axis_index("c")
    sid = lax.axis_index("s")
    wid = cid * NS + sid
    base = wid * EPT
    for i in range(STRIPE // L + 1):
        zbuf[pl.ds(i * L, L)] = jnp.zeros((L,), jnp.float32)
    for i in range(KE // L):
        ones_v[pl.ds(i * L, L)] = jnp.full((L,), 1.0, jnp.float32)
    pltpu.sync_copy(zbuf.at[pl.ds(0, STRIPE)], acc.at[pl.ds(sid * STRIPE, STRIPE)])
    plsc.subcore_barrier()

    def idx_src(q):
        return colp_hbm.at[pl.ds(pl.multiple_of(base + q * KE, KE), KE)]

    pltpu.async_copy(idx_src(0), col4.at[0], si[0])
    pltpu.async_copy(idx_src(1), col4.at[1], si[1])

    def body(i, carry):
        for b in range(4):
            v = i * 4 + b
            b2 = (b + 2) % 4
            pltpu.make_async_copy(idx_src(v), col4.at[b], si[b]).wait()
            pltpu.async_copy(ones_v, acc.at[col4.at[b]], ss[b], add=True)
            w = v + 2

            @pl.when(jnp.logical_and(w < CH, w >= 4))
            def _():
                pltpu.make_async_copy(ones_v, acc.at[col4.at[b2]], ss[b2]).wait()

            @pl.when(w < CH)
            def _():
                pltpu.async_copy(idx_src(w), col4.at[b2], si[b2])
        return carry

    lax.fori_loop(0, CH // 4, body, 0)
    for b in range(4):
        pltpu.make_async_copy(ones_v, acc.at[col4.at[b]], ss[b]).wait()
    plsc.subcore_barrier()
    off = pl.multiple_of(cid * NP + sid * STRIPE, 8)
    pltpu.sync_copy(acc.at[pl.ds(sid * STRIPE, STRIPE)], zbuf.at[pl.ds(0, STRIPE)])
    pltpu.sync_copy(zbuf.at[pl.ds(0, STRIPE)], out_hbm.at[pl.ds(off, STRIPE)])


@functools.cache
def _sc_deg_kernel():
    return pl.kernel(
        _deg_body,
        out_type=jax.ShapeDtypeStruct((NC * NP,), jnp.float32),
        mesh=_mesh(),
        scratch_types=[
            pltpu.VMEM((4, KE), jnp.int32),
            pltpu.VMEM((KE,), jnp.float32),
            pltpu.VMEM((STRIPE // L * L + L,), jnp.float32),
        ] + [pltpu.SemaphoreType.DMA] * 8 + [
            pltpu.VMEM_SHARED((NP,), jnp.float32),
        ],
    )


def _sc_deg(colp):
    return _sc_deg_kernel()(colp).reshape(NC, NP)


def _prop_body(u_hbm, colp_hbm, rowp_hbm, out_hbm, col8, row8,
               r0, r1, r2, r3, sg0, sg1, sg2, sg3, ss0, ss1, ss2, ss3,
               si0, si1, si2, si3, si4, si5, si6, si7, acc):
    rows = (r0, r1, r2, r3)
    sg = (sg0, sg1, sg2, sg3)
    ss = (ss0, ss1, ss2, ss3)
    si = (si0, si1, si2, si3, si4, si5, si6, si7)
    cid = lax.axis_index("c")
    sid = lax.axis_index("s")
    base = jnp.where(cid == 0, sid * EPT0, F0 + sid * EPT1)
    chn = jnp.where(cid == 0, CH0, CH1)
    for o in range(0, STRIPE, KE):
        sz = min(KE, STRIPE - o)
        pltpu.sync_copy(u_hbm.at[pl.ds(sid * STRIPE + o, sz)], r0.at[pl.ds(0, sz)])
        pltpu.sync_copy(r0.at[pl.ds(0, sz)], acc.at[pl.ds(sid * STRIPE + o, sz)])
    plsc.subcore_barrier()

    def cidx_src(q):
        return colp_hbm.at[pl.ds(pl.multiple_of(base + q * KE, KE), KE)]

    def ridx_src(q):
        return rowp_hbm.at[pl.ds(pl.multiple_of(base + q * KE, KE), KE)]

    for q in range(6):
        pltpu.async_copy(cidx_src(q), col8.at[q], si[q])
        pltpu.async_copy(ridx_src(q), row8.at[q], si[q])
    for v in range(2):
        pltpu.make_async_copy(cidx_src(v), col8.at[v], si[v]).wait()
        pltpu.make_async_copy(ridx_src(v), row8.at[v], si[v]).wait()
        pltpu.async_copy(u_hbm.at[col8.at[v]], rows[v], sg[v])

    def body(i, carry):
        for k in range(8):
            v = i * 8 + k
            b = k % 4
            b2 = (b + 2) % 4
            c2 = (k + 2) % 8
            c6 = (k + 6) % 8
            pltpu.make_async_copy(u_hbm.at[col8.at[k]], rows[b], sg[b]).wait()
            pltpu.async_copy(rows[b], acc.at[row8.at[k]], ss[b], add=True)
            w = v + 2

            @pl.when(jnp.logical_and(w < chn, w >= 4))
            def _():
                pltpu.make_async_copy(rows[b2], acc.at[row8.at[c2]], ss[b2]).wait()

            @pl.when(w < chn)
            def _():
                pltpu.make_async_copy(cidx_src(w), col8.at[c2], si[c2]).wait()
                pltpu.make_async_copy(ridx_src(w), row8.at[c2], si[c2]).wait()
                pltpu.async_copy(u_hbm.at[col8.at[c2]], rows[b2], sg[b2])

            q = v + 6

            @pl.when(q < chn)
            def _():
                pltpu.async_copy(cidx_src(q), col8.at[c6], si[c6])
                pltpu.async_copy(ridx_src(q), row8.at[c6], si[c6])
        return carry

    lax.fori_loop(0, chn // 8, body, 0)
    for j in range(4):
        pltpu.make_async_copy(rows[j], acc.at[row8.at[4 + j]], ss[j]).wait()
    plsc.subcore_barrier()
    for o in range(0, STRIPE, KE):
        sz = min(KE, STRIPE - o)
        pltpu.sync_copy(acc.at[pl.ds(sid * STRIPE + o, sz)], r0.at[pl.ds(0, sz)])
        pltpu.sync_copy(r0.at[pl.ds(0, sz)],
                        out_hbm.at[cid, pl.ds(sid * STRIPE + o, sz)])


@functools.cache
def _sc_prop_kernel():
    return pl.kernel(
        _prop_body,
        out_type=jax.ShapeDtypeStruct((NC, NP, D), jnp.float32),
        mesh=_mesh(),
        scratch_types=[
            pltpu.VMEM((NIDX, KE), jnp.int32),
            pltpu.VMEM((NIDX, KE), jnp.int32),
            pltpu.VMEM((KE, D), jnp.float32),
            pltpu.VMEM((KE, D), jnp.float32),
            pltpu.VMEM((KE, D), jnp.float32),
            pltpu.VMEM((KE, D), jnp.float32),
        ] + [pltpu.SemaphoreType.DMA] * 16 + [
            pltpu.VMEM_SHARED((NP, D), jnp.float32),
        ],
    )


def _sc_prop(u, colp, rowp):
    return _sc_prop_kernel()(u, colp, rowp)


BR = STRIPE
GRID = NP // BR


def _rows(i):
    return (i, 0)


def _const(*_):
    return (0, 0)


def _const3(*_):
    return (0, 0, 0)


def _rows3(i):
    return (0, i, 0)


def _rspec(width=D):
    return pl.BlockSpec((BR, width), _rows)


def _pspec():
    return pl.BlockSpec((NC, BR, D), _rows3)


def _wspec(shape):
    return pl.BlockSpec(shape, _const)


def _dot(a, b):
    return jnp.dot(a, b, preferred_element_type=jnp.float32)


def _pre_body(degT_r, x_r, w0_r, w1_r, w2_r, b0_r, b1_r, b2_r,
              dis_o, h0_o, uA_o, uB_o):
    d = degT_r[:, 0:1] + degT_r[:, 1:2] + 1.0
    dis = lax.rsqrt(d)
    dis_o[...] = dis
    xb = x_r[...]
    h0_o[...] = _dot(xb, w0_r[...]) + b0_r[...]
    uA_o[...] = dis * (_dot(xb, w1_r[...]) + b1_r[...])
    uB_o[...] = dis * (_dot(xb, w2_r[...]) + b2_r[...])


@functools.cache
def _k_pre():
    return pl.pallas_call(
        _pre_body,
        grid=(GRID,),
        in_specs=[pl.BlockSpec((BR, 2), _rows), _rspec(),
                  _wspec((D, D)), _wspec((D, D)), _wspec((D, D)),
                  _wspec((1, D)), _wspec((1, D)), _wspec((1, D))],
        out_specs=[pl.BlockSpec((BR, 1), _rows), _rspec(), _rspec(), _rspec()],
        out_shape=[jax.ShapeDtypeStruct((NP, 1), jnp.float32)] +
                  [jax.ShapeDtypeStruct((NP, D), jnp.float32)] * 3,
    )


def _comb_body(pA_r, pB_r, uA_r, uB_r, dis_r, h1_o, uB2_o):
    dis = dis_r[...]
    h1_o[...] = dis * (pA_r[0] + pA_r[1] - uA_r[...])
    uB2_o[...] = dis * dis * (pB_r[0] + pB_r[1] - uB_r[...])


@functools.cache
def _k_comb():
    return pl.pallas_call(
        _comb_body,
        grid=(GRID,),
        in_specs=[_pspec(), _pspec(), _rspec(), _rspec(),
                  pl.BlockSpec((BR, 1), _rows)],
        out_specs=[_rspec(), _rspec()],
        out_shape=[jax.ShapeDtypeStruct((NP, D), jnp.float32)] * 2,
    )


def _stats_body(h0_r, h1_r, pB2_r, uB2_r, dis_r, h2_o, s_o):
    i = pl.program_id(0)
    dis = dis_r[...]
    h2 = dis * (pB2_r[0] + pB2_r[1] - uB2_r[...])
    h2_o[...] = h2
    rowid = lax.broadcasted_iota(jnp.int32, (BR, 1), 0) + i * BR
    m = jnp.where(rowid < N, 1.0, 0.0)
    h0 = h0_r[...] * m
    h1 = h1_r[...] * m
    h2 = h2 * m
    srow = jnp.concatenate(
        [jnp.sum(h0, 0, keepdims=True), jnp.sum(h1, 0, keepdims=True),
         jnp.sum(h2, 0, keepdims=True)], axis=1)
    qrow = jnp.concatenate(
        [jnp.sum(h0 * h0, 0, keepdims=True), jnp.sum(h1 * h1, 0, keepdims=True),
         jnp.sum(h2 * h2, 0, keepdims=True)], axis=1)

    @pl.when(i == 0)
    def _():
        s_o[...] = jnp.zeros_like(s_o)

    s_o[0:1, :] += srow
    s_o[1:2, :] += qrow


@functools.cache
def _k_stats():
    return pl.pallas_call(
        _stats_body,
        grid=(GRID,),
        in_specs=[_rspec(), _rspec(), _pspec(), _rspec(),
                  pl.BlockSpec((BR, 1), _rows)],
        out_specs=[_rspec(), pl.BlockSpec((8, 3 * D), _const)],
        out_shape=[jax.ShapeDtypeStruct((NP, D), jnp.float32),
                   jax.ShapeDtypeStruct((8, 3 * D), jnp.float32)],
    )


def _l2_body(h0_r, h1_r, h2_r, s_r, dis_r, w0_r, w1_r, w2_r,
             b0_r, b1_r, b2_r, g_r, bb_r, g0_o, vC_o, vD_o):
    inv_n = 1.0 / N
    mean = s_r[0:1, :] * inv_n
    var = s_r[1:2, :] * inv_n - mean * mean
    inv = lax.rsqrt(var + 1e-5)
    scale = inv * g_r[...]
    shift = bb_r[...] - mean * scale
    z0 = jnp.maximum(h0_r[...] * scale[:, 0:D] + shift[:, 0:D], 0.0)
    z1 = jnp.maximum(h1_r[...] * scale[:, D:2 * D] + shift[:, D:2 * D], 0.0)
    z2 = jnp.maximum(h2_r[...] * scale[:, 2 * D:] + shift[:, 2 * D:], 0.0)
    z = jnp.concatenate([z0, z1, z2], axis=1)
    dis = dis_r[...]
    g0_o[...] = _dot(z, w0_r[...]) + b0_r[...]
    vC_o[...] = dis * (_dot(z, w1_r[...]) + b1_r[...])
    vD_o[...] = dis * (_dot(z, w2_r[...]) + b2_r[...])


@functools.cache
def _k_l2():
    return pl.pallas_call(
        _l2_body,
        grid=(GRID,),
        in_specs=[_rspec(), _rspec(), _rspec(),
                  pl.BlockSpec((8, 3 * D), _const),
                  pl.BlockSpec((BR, 1), _rows),
                  _wspec((3 * D, D)), _wspec((3 * D, D)), _wspec((3 * D, D)),
                  _wspec((1, D)), _wspec((1, D)), _wspec((1, D)),
                  _wspec((1, 3 * D)), _wspec((1, 3 * D))],
        out_specs=[_rspec(), _rspec(), _rspec()],
        out_shape=[jax.ShapeDtypeStruct((NP, D), jnp.float32)] * 3,
    )


def _final_body(g0_r, g1_r, pD2_r, uD2_r, dis_r, w0_r, w1_r, w2_r, bf_r, out_o):
    dis = dis_r[...]
    g2 = dis * (pD2_r[0] + pD2_r[1] - uD2_r[...])
    out_o[...] = (_dot(g0_r[...], w0_r[...]) + _dot(g1_r[...], w1_r[...]) +
                  _dot(g2, w2_r[...]) + bf_r[...])


@functools.cache
def _k_final():
    return pl.pallas_call(
        _final_body,
        grid=(GRID,),
        in_specs=[_rspec(), _rspec(), _pspec(), _rspec(),
                  pl.BlockSpec((BR, 1), _rows),
                  _wspec((D, D)), _wspec((D, D)), _wspec((D, D)),
                  _wspec((1, D))],
        out_specs=_rspec(),
        out_shape=jax.ShapeDtypeStruct((NP, D), jnp.float32),
    )


def kernel(x, edge_index, W0_0, b0_0, W0_1, b0_1, W0_2, b0_2, bn_g, bn_b,
           W1_0, b1_0, W1_1, b1_1, W1_2, b1_2, Wf, bf):
    row = edge_index[0]
    col = edge_index[1]
    pad_idx = jnp.full((EPAD - E,), N, jnp.int32)
    rowp = jnp.concatenate([row, pad_idx])
    colp = jnp.concatenate([col, pad_idx])

    degp = _sc_deg(colp)
    degT = degp.T
    xp = jnp.pad(x, ((0, NP - N), (0, 0)))

    dis, h0, uA, uB = _k_pre()(
        degT, xp, W0_0.T, W0_1.T, W0_2.T,
        b0_0[None, :], b0_1[None, :], b0_2[None, :])

    pA = _sc_prop(uA, colp, rowp)
    pB1 = _sc_prop(uB, colp, rowp)
    h1, uB2 = _k_comb()(pA, pB1, uA, uB, dis)
    pB2 = _sc_prop(uB2, colp, rowp)
    h2, S = _k_stats()(h0, h1, pB2, uB2, dis)

    g0, vC, vD = _k_l2()(
        h0, h1, h2, S, dis, W1_0.T, W1_1.T, W1_2.T,
        b1_0[None, :], b1_1[None, :], b1_2[None, :],
        bn_g[None, :], bn_b[None, :])

    pC = _sc_prop(vC, colp, rowp)
    pD1 = _sc_prop(vD, colp, rowp)
    g1, uD2 = _k_comb()(pC, pD1, vC, vD, dis)
    pD2 = _sc_prop(uD2, colp, rowp)

    out = _k_final()(g0, g1, pD2, uD2, dis,
                     Wf[:, 0:D].T, Wf[:, D:2 * D].T, Wf[:, 2 * D:].T,
                     bf[None, :])
    return out[:N]


# speedup vs baseline: 1.0020x; 1.0020x over previous
import functools

import jax
import jax.numpy as jnp
from jax import lax
from jax.experimental import pallas as pl
from jax.experimental.pallas import tpu as pltpu
from jax.experimental.pallas import tpu_sc as plsc

N = 10000
D = 128
NP = 10112
E = 320000
NC, NS, L = 2, 16, 16
NW = NC * NS
KE = 64
CH = 160
EPT = CH * KE
EPAD = EPT * NW
STRIPE = NP // NS
NBUF = 4
NIDX = 8
CH0 = 296
CH1 = 24
EPT0 = CH0 * KE
EPT1 = CH1 * KE
F0 = EPT0 * NS


@functools.cache
def _mesh():
    return plsc.VectorSubcoreMesh(
        core_axis_name="c", subcore_axis_name="s", num_cores=NC, num_subcores=NS)


def _deg_body(colp_hbm, out_hbm, col4, ones_v, zbuf, si0, si1, si2, si3,
              ss0, ss1, ss2, ss3, acc):
    si = (si0, si1, si2, si3)
    ss = (ss0, ss1, ss2, ss3)
    cid = lax.axis_index("c")
    sid = lax.axis_index("s")
    wid = cid * NS + sid
    base = wid * EPT
    for i in range(STRIPE // L + 1):
        zbuf[pl.ds(i * L, L)] = jnp.zeros((L,), jnp.float32)
    for i in range(KE // L):
        ones_v[pl.ds(i * L, L)] = jnp.full((L,), 1.0, jnp.float32)
    pltpu.sync_copy(zbuf.at[pl.ds(0, STRIPE)], acc.at[pl.ds(sid * STRIPE, STRIPE)])
    plsc.subcore_barrier()

    def idx_src(q):
        return colp_hbm.at[pl.ds(pl.multiple_of(base + q * KE, KE), KE)]

    pltpu.async_copy(idx_src(0), col4.at[0], si[0])
    pltpu.async_copy(idx_src(1), col4.at[1], si[1])

    def body(i, carry):
        for b in range(4):
            v = i * 4 + b
            b2 = (b + 2) % 4
            pltpu.make_async_copy(idx_src(v), col4.at[b], si[b]).wait()
            pltpu.async_copy(ones_v, acc.at[col4.at[b]], ss[b], add=True)
            w = v + 2

            @pl.when(jnp.logical_and(w < CH, w >= 4))
            def _():
                pltpu.make_async_copy(ones_v, acc.at[col4.at[b2]], ss[b2]).wait()

            @pl.when(w < CH)
            def _():
                pltpu.async_copy(idx_src(w), col4.at[b2], si[b2])
        return carry

    lax.fori_loop(0, CH // 4, body, 0)
    for b in range(4):
        pltpu.make_async_copy(ones_v, acc.at[col4.at[b]], ss[b]).wait()
    plsc.subcore_barrier()
    off = pl.multiple_of(cid * NP + sid * STRIPE, 8)
    pltpu.sync_copy(acc.at[pl.ds(sid * STRIPE, STRIPE)], zbuf.at[pl.ds(0, STRIPE)])
    pltpu.sync_copy(zbuf.at[pl.ds(0, STRIPE)], out_hbm.at[pl.ds(off, STRIPE)])


@functools.cache
def _sc_deg_kernel():
    return pl.kernel(
        _deg_body,
        out_type=jax.ShapeDtypeStruct((NC * NP,), jnp.float32),
        mesh=_mesh(),
        scratch_types=[
            pltpu.VMEM((4, KE), jnp.int32),
            pltpu.VMEM((KE,), jnp.float32),
            pltpu.VMEM((STRIPE // L * L + L,), jnp.float32),
        ] + [pltpu.SemaphoreType.DMA] * 8 + [
            pltpu.VMEM_SHARED((NP,), jnp.float32),
        ],
    )


def _sc_deg(colp):
    return _sc_deg_kernel()(colp).reshape(NC, NP)


def _prop_body(u_hbm, colp_hbm, rowp_hbm, out_hbm, col8, row8,
               r0, r1, r2, r3, sg0, sg1, sg2, sg3, ss0, ss1, ss2, ss3,
               si0, si1, si2, si3, si4, si5, si6, si7, acc):
    rows = (r0, r1, r2, r3)
    sg = (sg0, sg1, sg2, sg3)
    ss = (ss0, ss1, ss2, ss3)
    si = (si0, si1, si2, si3, si4, si5, si6, si7)
    cid = lax.axis_index("c")
    sid = lax.axis_index("s")
    base = jnp.where(cid == 0, sid * EPT0, F0 + sid * EPT1)
    chn = jnp.where(cid == 0, CH0, CH1)
    for o in range(0, STRIPE, KE):
        sz = min(KE, STRIPE - o)
        pltpu.sync_copy(u_hbm.at[pl.ds(sid * STRIPE + o, sz)], r0.at[pl.ds(0, sz)])
        pltpu.sync_copy(r0.at[pl.ds(0, sz)], acc.at[pl.ds(sid * STRIPE + o, sz)])
    plsc.subcore_barrier()

    def cidx_src(q):
        return colp_hbm.at[pl.ds(pl.multiple_of(base + q * KE, KE), KE)]

    def ridx_src(q):
        return rowp_hbm.at[pl.ds(pl.multiple_of(base + q * KE, KE), KE)]

    for q in range(6):
        pltpu.async_copy(cidx_src(q), col8.at[q], si[q])
        pltpu.async_copy(ridx_src(q), row8.at[q], si[q])
    for v in range(2):
        pltpu.make_async_copy(cidx_src(v), col8.at[v], si[v]).wait()
        pltpu.make_async_copy(ridx_src(v), row8.at[v], si[v]).wait()
        pltpu.async_copy(u_hbm.at[col8.at[v]], rows[v], sg[v])

    def body(i, carry):
        for k in range(8):
            v = i * 8 + k
            b = k % 4
            b2 = (b + 2) % 4
            c2 = (k + 2) % 8
            c6 = (k + 6) % 8
            pltpu.make_async_copy(u_hbm.at[col8.at[k]], rows[b], sg[b]).wait()
            pltpu.async_copy(rows[b], acc.at[row8.at[k]], ss[b], add=True)
            w = v + 2

            @pl.when(jnp.logical_and(w < chn, w >= 4))
            def _():
                pltpu.make_async_copy(rows[b2], acc.at[row8.at[c2]], ss[b2]).wait()

            @pl.when(w < chn)
            def _():
                pltpu.make_async_copy(cidx_src(w), col8.at[c2], si[c2]).wait()
                pltpu.make_async_copy(ridx_src(w), row8.at[c2], si[c2]).wait()
                pltpu.async_copy(u_hbm.at[col8.at[c2]], rows[b2], sg[b2])

            q = v + 6

            @pl.when(q < chn)
            def _():
                pltpu.async_copy(cidx_src(q), col8.at[c6], si[c6])
                pltpu.async_copy(ridx_src(q), row8.at[c6], si[c6])
        return carry

    lax.fori_loop(0, chn // 8, body, 0)
    for j in range(4):
        pltpu.make_async_copy(rows[j], acc.at[row8.at[4 + j]], ss[j]).wait()
    plsc.subcore_barrier()
    for o in range(0, STRIPE, KE):
        sz = min(KE, STRIPE - o)
        pltpu.sync_copy(acc.at[pl.ds(sid * STRIPE + o, sz)], r0.at[pl.ds(0, sz)])
        pltpu.sync_copy(r0.at[pl.ds(0, sz)],
                        out_hbm.at[cid, pl.ds(sid * STRIPE + o, sz)])


@functools.cache
def _sc_prop_kernel():
    return pl.kernel(
        _prop_body,
        out_type=jax.ShapeDtypeStruct((NC, NP, D), jnp.float32),
        mesh=_mesh(),
        scratch_types=[
            pltpu.VMEM((NIDX, KE), jnp.int32),
            pltpu.VMEM((NIDX, KE), jnp.int32),
            pltpu.VMEM((KE, D), jnp.float32),
            pltpu.VMEM((KE, D), jnp.float32),
            pltpu.VMEM((KE, D), jnp.float32),
            pltpu.VMEM((KE, D), jnp.float32),
        ] + [pltpu.SemaphoreType.DMA] * 16 + [
            pltpu.VMEM_SHARED((NP, D), jnp.float32),
        ],
    )


def _sc_prop(u, colp, rowp):
    return _sc_prop_kernel()(u, colp, rowp)


BR = STRIPE
GRID = NP // BR


def _rows(i):
    return (i, 0)


def _const(*_):
    return (0, 0)


def _const3(*_):
    return (0, 0, 0)


def _rows3(i):
    return (0, i, 0)


def _rspec(width=D):
    return pl.BlockSpec((BR, width), _rows)


def _pspec():
    return pl.BlockSpec((NC, BR, D), _rows3)


def _wspec(shape):
    return pl.BlockSpec(shape, _const)


def _dot(a, b):
    return jnp.dot(a, b, preferred_element_type=jnp.float32)


def _pre_body(degT_r, x_r, w0_r, w1_r, w2_r, b0_r, b1_r, b2_r,
              dis_o, h0_o, uA_o, uB_o):
    d = degT_r[:, 0:1] + degT_r[:, 1:2] + 1.0
    dis = lax.rsqrt(d)
    dis_o[...] = dis
    xb = x_r[...]
    h0_o[...] = _dot(xb, w0_r[...]) + b0_r[...]
    uA_o[...] = dis * (_dot(xb, w1_r[...]) + b1_r[...])
    uB_o[...] = dis * (_dot(xb, w2_r[...]) + b2_r[...])


@functools.cache
def _k_pre():
    return pl.pallas_call(
        _pre_body,
        grid=(GRID,),
        in_specs=[pl.BlockSpec((BR, 2), _rows), _rspec(),
                  _wspec((D, D)), _wspec((D, D)), _wspec((D, D)),
                  _wspec((1, D)), _wspec((1, D)), _wspec((1, D))],
        out_specs=[pl.BlockSpec((BR, 1), _rows), _rspec(), _rspec(), _rspec()],
        out_shape=[jax.ShapeDtypeStruct((NP, 1), jnp.float32)] +
                  [jax.ShapeDtypeStruct((NP, D), jnp.float32)] * 3,
    )


def _comb_body(pA_r, pB_r, uA_r, uB_r, dis_r, h1_o, uB2_o):
    dis = dis_r[...]
    h1_o[...] = dis * (pA_r[0] + pA_r[1] - uA_r[...])
    uB2_o[...] = dis * dis * (pB_r[0] + pB_r[1] - uB_r[...])


@functools.cache
def _k_comb():
    return pl.pallas_call(
        _comb_body,
        grid=(GRID,),
        in_specs=[_pspec(), _pspec(), _rspec(), _rspec(),
                  pl.BlockSpec((BR, 1), _rows)],
        out_specs=[_rspec(), _rspec()],
        out_shape=[jax.ShapeDtypeStruct((NP, D), jnp.float32)] * 2,
    )


def _stats_body(h0_r, h1_r, pB2_r, uB2_r, dis_r, h2_o, s_o):
    i = pl.program_id(0)
    dis = dis_r[...]
    h2 = dis * (pB2_r[0] + pB2_r[1] - uB2_r[...])
    h2_o[...] = h2
    rowid = lax.broadcasted_iota(jnp.int32, (BR, 1), 0) + i * BR
    m = jnp.where(rowid < N, 1.0, 0.0)
    h0 = h0_r[...] * m
    h1 = h1_r[...] * m
    h2 = h2 * m
    srow = jnp.concatenate(
        [jnp.sum(h0, 0, keepdims=True), jnp.sum(h1, 0, keepdims=True),
         jnp.sum(h2, 0, keepdims=True)], axis=1)
    qrow = jnp.concatenate(
        [jnp.sum(h0 * h0, 0, keepdims=True), jnp.sum(h1 * h1, 0, keepdims=True),
         jnp.sum(h2 * h2, 0, keepdims=True)], axis=1)

    @pl.when(i == 0)
    def _():
        s_o[...] = jnp.zeros_like(s_o)

    s_o[0:1, :] += srow
    s_o[1:2, :] += qrow


@functools.cache
def _k_stats():
    return pl.pallas_call(
        _stats_body,
        grid=(GRID,),
        in_specs=[_rspec(), _rspec(), _pspec(), _rspec(),
                  pl.BlockSpec((BR, 1), _rows)],
        out_specs=[_rspec(), pl.BlockSpec((8, 3 * D), _const)],
        out_shape=[jax.ShapeDtypeStruct((NP, D), jnp.float32),
                   jax.ShapeDtypeStruct((8, 3 * D), jnp.float32)],
    )


def _l2_body(h0_r, h1_r, h2_r, s_r, dis_r, w0_r, w1_r, w2_r,
             b0_r, b1_r, b2_r, g_r, bb_r, g0_o, vC_o, vD_o):
    inv_n = 1.0 / N
    mean = s_r[0:1, :] * inv_n
    var = s_r[1:2, :] * inv_n - mean * mean
    inv = lax.rsqrt(var + 1e-5)
    scale = inv * g_r[...]
    shift = bb_r[...] - mean * scale
    z0 = jnp.maximum(h0_r[...] * scale[:, 0:D] + shift[:, 0:D], 0.0)
    z1 = jnp.maximum(h1_r[...] * scale[:, D:2 * D] + shift[:, D:2 * D], 0.0)
    z2 = jnp.maximum(h2_r[...] * scale[:, 2 * D:] + shift[:, 2 * D:], 0.0)
    z = jnp.concatenate([z0, z1, z2], axis=1)
    dis = dis_r[...]
    g0_o[...] = _dot(z, w0_r[...]) + b0_r[...]
    vC_o[...] = dis * (_dot(z, w1_r[...]) + b1_r[...])
    vD_o[...] = dis * (_dot(z, w2_r[...]) + b2_r[...])


@functools.cache
def _k_l2():
    return pl.pallas_call(
        _l2_body,
        grid=(GRID,),
        in_specs=[_rspec(), _rspec(), _rspec(),
                  pl.BlockSpec((8, 3 * D), _const),
                  pl.BlockSpec((BR, 1), _rows),
                  _wspec((3 * D, D)), _wspec((3 * D, D)), _wspec((3 * D, D)),
                  _wspec((1, D)), _wspec((1, D)), _wspec((1, D)),
                  _wspec((1, 3 * D)), _wspec((1, 3 * D))],
        out_specs=[_rspec(), _rspec(), _rspec()],
        out_shape=[jax.ShapeDtypeStruct((NP, D), jnp.float32)] * 3,
    )


def _final_body(g0_r, g1_r, pD2_r, uD2_r, dis_r, w0_r, w1_r, w2_r, bf_r, out_o):
    dis = dis_r[...]
    g2 = dis * (pD2_r[0] + pD2_r[1] - uD2_r[...])
    out_o[...] = (_dot(g0_r[...], w0_r[...]) + _dot(g1_r[...], w1_r[...]) +
                  _dot(g2, w2_r[...]) + bf_r[...])


@functools.cache
def _k_final():
    return pl.pallas_call(
        _final_body,
        grid=(GRID,),
        in_specs=[_rspec(), _rspec(), _pspec(), _rspec(),
                  pl.BlockSpec((BR, 1), _rows),
                  _wspec((D, D)), _wspec((D, D)), _wspec((D, D)),
                  _wspec((1, D))],
        out_specs=_rspec(),
        out_shape=jax.ShapeDtypeStruct((NP, D), jnp.float32),
    )


def kernel(x, edge_index, W0_0, b0_0, W0_1, b0_1, W0_2, b0_2, bn_g, bn_b,
           W1_0, b1_0, W1_1, b1_1, W1_2, b1_2, Wf, bf):
    row = edge_index[0]
    col = edge_index[1]
    pad_idx = jnp.full((EPAD - E,), N, jnp.int32)
    rowp = jnp.concatenate([row, pad_idx])
    colp = jnp.concatenate([col, pad_idx])

    degp = _sc_deg(colp)
    degT = degp.T
    xp = jnp.pad(x, ((0, NP - N), (0, 0)))

    dis, h0, uA, uB = _k_pre()(
        degT, xp, W0_0.T, W0_1.T, W0_2.T,
        b0_0[None, :], b0_1[None, :], b0_2[None, :])

    pA = _sc_prop(uA, colp, rowp)
    pB1 = _sc_prop(uB, colp, rowp)
    h1, uB2 = _k_comb()(pA, pB1, uA, uB, dis)
    pB2 = _sc_prop(uB2, colp, rowp)
    h2, S = _k_stats()(h0, h1, pB2, uB2, dis)

    g0, vC, vD = _k_l2()(
        h0, h1, h2, S, dis, W1_0.T, W1_1.T, W1_2.T,
        b1_0[None, :], b1_1[None, :], b1_2[None, :],
        bn_g[None, :], bn_b[None, :])

    pC = _sc_prop(vC, colp, rowp)
    pD1 = _sc_prop(vD, colp, rowp)
    g1, uD2 = _k_comb()(pC, pD1, vC, vD, dis)
    pD2 = _sc_prop(uD2, colp, rowp)

    out = _k_final()(g0, g1, pD2, uD2, dis,
                     Wf[:, 0:D].T, Wf[:, D:2 * D].T, Wf[:, 2 * D:].T,
                     bf[None, :])
    return out[:N]
